# baseline (speedup 1.0000x reference)
import functools

import jax
import jax.numpy as jnp
from jax import lax
from jax.experimental import pallas as pl
from jax.experimental.pallas import tpu as pltpu
from jax.experimental.pallas import tpu_sc as plsc


@functools.cache
def _make_gather(V, D, B, chunk):
  info = plsc.get_sparse_core_info()
  NC, NS = info.num_cores, info.num_subcores
  NW = NC * NS
  assert B % NW == 0
  b_per_w = B // NW
  assert b_per_w % chunk == 0
  n_chunks = b_per_w // chunk
  mesh = plsc.VectorSubcoreMesh(core_axis_name="c", subcore_axis_name="s")

  @functools.partial(
      pl.kernel,
      mesh=mesh,
      out_type=jax.ShapeDtypeStruct((B, 1, D), jnp.float32),
      scratch_types=[
          pltpu.VMEM((b_per_w,), jnp.int32),
          pltpu.VMEM((n_chunks, chunk, D), jnp.float32),
          pltpu.SemaphoreType.DMA,
          pltpu.SemaphoreType.DMA,
      ],
  )
  def k(idx_hbm, table_hbm, out_hbm, idx_v, rows_v, gsem, osem):
    wid = lax.axis_index("s") * NC + lax.axis_index("c")
    base = wid * b_per_w
    pltpu.sync_copy(idx_hbm.at[pl.ds(base, b_per_w)], idx_v)
    gathers = [
        pltpu.async_copy(
            table_hbm.at[idx_v.at[pl.ds(j * chunk, chunk)]], rows_v.at[j], gsem
        )
        for j in range(n_chunks)
    ]
    outs = []
    for j in range(n_chunks):
      gathers[j].wait()
      outs.append(
          pltpu.async_copy(
              rows_v.at[j], out_hbm.at[pl.ds(base + j * chunk, chunk), 0], osem
          )
      )
    for o in outs:
      o.wait()

  return k


def kernel(class_ids, weight):
  B = class_ids.shape[0]
  V, D = weight.shape
  g = _make_gather(V, D, B, 128)
  return g(class_ids.astype(jnp.int32), weight)


# speedup vs baseline: 1.0224x; 1.0224x over previous
import functools

import jax
import jax.numpy as jnp
from jax import lax
from jax.experimental import pallas as pl
from jax.experimental.pallas import tpu as pltpu
from jax.experimental.pallas import tpu_sc as plsc


@functools.cache
def _make_gather(V, D, B, chunk):
  info = plsc.get_sparse_core_info()
  NC, NS = info.num_cores, info.num_subcores
  NW = NC * NS
  assert B % NW == 0
  b_per_w = B // NW
  assert b_per_w % chunk == 0
  n_chunks = b_per_w // chunk
  mesh = plsc.VectorSubcoreMesh(core_axis_name="c", subcore_axis_name="s")

  @functools.partial(
      pl.kernel,
      mesh=mesh,
      out_type=jax.ShapeDtypeStruct((B, 1, D), jnp.float32),
      scratch_types=[
          pltpu.VMEM((b_per_w,), jnp.int32),
          pltpu.VMEM((n_chunks, chunk, D), jnp.float32),
          pltpu.SemaphoreType.DMA,
          pltpu.SemaphoreType.DMA,
      ],
  )
  def k(idx_hbm, table_hbm, out_hbm, idx_v, rows_v, gsem, osem):
    wid = lax.axis_index("s") * NC + lax.axis_index("c")
    base = wid * b_per_w
    pltpu.sync_copy(idx_hbm.at[pl.ds(base, b_per_w)], idx_v)
    gathers = [
        pltpu.async_copy(
            table_hbm.at[idx_v.at[pl.ds(j * chunk, chunk)]], rows_v.at[j], gsem
        )
        for j in range(n_chunks)
    ]
    outs = []
    for j in range(n_chunks):
      gathers[j].wait()
      outs.append(
          pltpu.async_copy(
              rows_v.at[j], out_hbm.at[pl.ds(base + j * chunk, chunk), 0], osem
          )
      )
    for o in outs:
      o.wait()

  return k


def kernel(class_ids, weight):
  B = class_ids.shape[0]
  V, D = weight.shape
  g = _make_gather(V, D, B, 512)
  return g(class_ids.astype(jnp.int32), weight)
